# Initial kernel scaffold; baseline (speedup 1.0000x reference)
import jax
import jax.numpy as jnp
from jax.experimental import pallas as pl


def kernel(user, item, user_emb, item_emb, user_topk, item_topk, W1, b1, W2, b2, W3, b3):
    raise NotImplementedError("write your pallas kernel here")


# speedup vs baseline: 2.2641x; 2.2641x over previous
import functools

import jax
import jax.numpy as jnp
from jax import lax
from jax.experimental import pallas as pl
from jax.experimental.pallas import tpu as pltpu
from jax.experimental.pallas import tpu_sc as plsc

_EMB = 32
_K = 5


def _sc_gather(user, item, user_emb, item_emb, user_topk, item_topk):
    B = user.shape[0]
    info = plsc.get_sparse_core_info()
    NC, NS = info.num_cores, info.num_subcores
    NW = NC * NS
    bpw = B // NW
    npw = bpw * _K

    mesh = plsc.VectorSubcoreMesh(core_axis_name="c", subcore_axis_name="s")

    @functools.partial(
        pl.kernel,
        out_type=(
            jax.ShapeDtypeStruct((B, _EMB), jnp.float32),
            jax.ShapeDtypeStruct((_K * B, _EMB), jnp.float32),
            jax.ShapeDtypeStruct((B, _EMB), jnp.float32),
            jax.ShapeDtypeStruct((_K * B, _EMB), jnp.float32),
        ),
        mesh=mesh,
        scratch_types=[
            pltpu.VMEM((bpw,), jnp.int32),
            pltpu.VMEM((bpw,), jnp.int32),
            pltpu.VMEM((npw,), jnp.int32),
            pltpu.VMEM((npw,), jnp.int32),
            pltpu.VMEM((bpw, _EMB), jnp.float32),
            pltpu.VMEM((npw, _EMB), jnp.float32),
            pltpu.SemaphoreType.DMA,
        ],
        compiler_params=pltpu.CompilerParams(use_tc_tiling_on_sc=False),
    )
    def k(user_h, item_h, uemb_h, iemb_h, utopk_h, itopk_h,
          out_ut, out_un, out_it, out_in,
          uid_v, iid_v, tidx_v, nid_v, targ_v, neib_v, sem):
        wid = lax.axis_index("s") * NC + lax.axis_index("c")
        base = wid * bpw
        pltpu.sync_copy(user_h.at[pl.ds(base, bpw)], uid_v)
        pltpu.sync_copy(item_h.at[pl.ds(base, bpw)], iid_v)

        def side(id_v, topk_h, emb_h, out_t, out_n):
            def build(c, carry):
                idk = id_v[pl.ds(c * 16, 16)] * _K
                for kk in range(_K):
                    tidx_v[pl.ds(kk * bpw + c * 16, 16)] = idk + kk
                return carry

            lax.fori_loop(0, bpw // 16, build, 0)

            pltpu.async_copy(emb_h.at[id_v], targ_v, sem).wait()
            pltpu.sync_copy(targ_v, out_t.at[pl.ds(base, bpw)])

            pltpu.async_copy(topk_h.at[tidx_v], nid_v, sem).wait()
            pltpu.async_copy(emb_h.at[nid_v], neib_v, sem).wait()
            for kk in range(_K):
                pltpu.sync_copy(neib_v.at[pl.ds(kk * bpw, bpw), :],
                                out_n.at[pl.ds(kk * B + base, bpw)])

        side(uid_v, utopk_h, uemb_h, out_ut, out_un)
        side(iid_v, itopk_h, iemb_h, out_it, out_in)

    return k(user, item, user_emb, item_emb,
             user_topk.reshape(-1), item_topk.reshape(-1))


def _tc_mlp(u_targ, u_nei, i_targ, i_nei,
            W1ut, W1un, W1it, W1in, b1, W2, b2, W3, b3):
    B = u_targ.shape[0]
    bB = 2048
    NT = _EMB
    NN = _K * _EMB
    H1 = W2.shape[0]
    H2 = W2.shape[1]

    def body(ut_ref, un_ref, it_ref, in_ref,
             w1ut_ref, w1un_ref, w1it_ref, w1in_ref,
             b1_ref, w2_ref, b2_ref, w3_ref, b3_ref, o_ref):
        h1 = jnp.dot(ut_ref[...], w1ut_ref[...], preferred_element_type=jnp.float32)
        h1 = h1 + jnp.dot(it_ref[...], w1it_ref[...], preferred_element_type=jnp.float32)
        for kk in range(_K):
            wu = w1un_ref[pl.ds(kk * _EMB, _EMB), :]
            wi = w1in_ref[pl.ds(kk * _EMB, _EMB), :]
            h1 = h1 + jnp.dot(un_ref[kk], wu, preferred_element_type=jnp.float32)
            h1 = h1 + jnp.dot(in_ref[kk], wi, preferred_element_type=jnp.float32)
        h1 = jnp.maximum(h1 + b1_ref[...], 0.0)
        h2 = jnp.dot(h1, w2_ref[...], preferred_element_type=jnp.float32)
        h2 = jnp.maximum(h2 + b2_ref[...], 0.0)
        o = jnp.dot(h2, w3_ref[...], preferred_element_type=jnp.float32)
        o_ref[...] = o + b3_ref[0, 0]

    return pl.pallas_call(
        body,
        grid=(B // bB,),
        in_specs=[
            pl.BlockSpec((bB, NT), lambda i: (i, 0)),
            pl.BlockSpec((_K, bB, NT), lambda i: (0, i, 0)),
            pl.BlockSpec((bB, NT), lambda i: (i, 0)),
            pl.BlockSpec((_K, bB, NT), lambda i: (0, i, 0)),
            pl.BlockSpec((NT, H1), lambda i: (0, 0)),
            pl.BlockSpec((NN, H1), lambda i: (0, 0)),
            pl.BlockSpec((NT, H1), lambda i: (0, 0)),
            pl.BlockSpec((NN, H1), lambda i: (0, 0)),
            pl.BlockSpec((1, H1), lambda i: (0, 0)),
            pl.BlockSpec((H1, H2), lambda i: (0, 0)),
            pl.BlockSpec((1, H2), lambda i: (0, 0)),
            pl.BlockSpec((H2, 1), lambda i: (0, 0)),
            pl.BlockSpec((1, 1), lambda i: (0, 0)),
        ],
        out_specs=pl.BlockSpec((bB, 1), lambda i: (i, 0)),
        out_shape=jax.ShapeDtypeStruct((B, 1), jnp.float32),
        compiler_params=pltpu.CompilerParams(
            dimension_semantics=("parallel",)),
    )(u_targ, u_nei, i_targ, i_nei,
      W1ut, W1un, W1it, W1in, b1, W2, b2, W3, b3)


def kernel(user, item, user_emb, item_emb, user_topk, item_topk,
           W1, b1, W2, b2, W3, b3):
    B = user.shape[0]
    user = user.astype(jnp.int32)
    item = item.astype(jnp.int32)
    user_topk = user_topk.astype(jnp.int32)
    item_topk = item_topk.astype(jnp.int32)

    u_targ, u_nei, i_targ, i_nei = _sc_gather(
        user, item, user_emb, item_emb, user_topk, item_topk)

    NN = _K * _EMB
    W1ut = W1[:_EMB]
    W1un = W1[_EMB:_EMB + NN]
    W1it = W1[_EMB + NN:2 * _EMB + NN]
    W1in = W1[2 * _EMB + NN:]
    out = _tc_mlp(u_targ, u_nei.reshape(_K, B, _EMB),
                  i_targ, i_nei.reshape(_K, B, _EMB),
                  W1ut, W1un, W1it, W1in,
                  b1.reshape(1, -1), W2, b2.reshape(1, -1),
                  W3, b3.reshape(1, 1))
    return out.reshape(B)
